# Initial kernel scaffold; baseline (speedup 1.0000x reference)
import jax
import jax.numpy as jnp
from jax.experimental import pallas as pl


def kernel(x, pos_edge_index, neg_edge_index, W1, b1, W2, b2):
    raise NotImplementedError("write your pallas kernel here")


# speedup vs baseline: 20.6324x; 20.6324x over previous
import functools

import jax
import jax.numpy as jnp
from jax import lax
from jax.experimental import pallas as pl
from jax.experimental.pallas import tpu as pltpu
from jax.experimental.pallas import tpu_sc as plsc

N_NODES = 10000
D_FEAT = 128
HIDDEN = 16
E1 = 320000
E2 = 640000

NC = 2
NS = 16
NW = NC * NS
K = 128
SINK = N_NODES
NP = 10240
RPT = NP // NS

RPW1 = 80
STEPS1 = NW * RPW1
EP1 = STEPS1 * K
RPW2 = 160
STEPS2 = NW * RPW2
EP2 = STEPS2 * K
EPW2 = RPW2 * K

_mesh = plsc.VectorSubcoreMesh(core_axis_name="c", subcore_axis_name="s")
_sc_params = pltpu.CompilerParams(use_tc_tiling_on_sc=False, needs_layout_passes=False)


@functools.partial(
    pl.kernel,
    mesh=_mesh,
    compiler_params=_sc_params,
    out_type=jax.ShapeDtypeStruct((NC, NP, HIDDEN), jnp.float32),
    scratch_types=[
        pltpu.VMEM((RPW1, K), jnp.int32),
        pltpu.VMEM((K, HIDDEN), jnp.float32),
        pltpu.VMEM((RPT, HIDDEN), jnp.float32),
        pltpu.VMEM_SHARED((NP, HIDDEN), jnp.float32),
    ],
)
def _deg_sc(dst_hbm, out_hbm, dstb, ones, zbuf, acc):
    c = lax.axis_index("c")
    s = lax.axis_index("s")
    wid = s * NC + c
    pltpu.sync_copy(dst_hbm.at[pl.ds(wid * RPW1, RPW1)], dstb)

    one_v = jnp.ones((HIDDEN,), jnp.float32)
    zero_v = jnp.zeros((HIDDEN,), jnp.float32)

    def fill_ones(i, carry):
        ones[i, :] = one_v
        return carry

    lax.fori_loop(0, K, fill_ones, 0)

    def fill_zero(i, carry):
        zbuf[i, :] = zero_v
        return carry

    lax.fori_loop(0, RPT, fill_zero, 0)
    base = s * RPT
    pltpu.sync_copy(zbuf, acc.at[pl.ds(base, RPT)])
    plsc.subcore_barrier()

    def body(j, carry):
        pltpu.sync_copy(ones, acc.at[dstb.at[j]], add=True)
        return carry

    lax.fori_loop(0, RPW1, body, 0)
    plsc.subcore_barrier()
    pltpu.sync_copy(acc.at[pl.ds(base, RPT)], out_hbm.at[c, pl.ds(base, RPT)])


@functools.partial(
    pl.kernel,
    mesh=_mesh,
    compiler_params=_sc_params,
    out_type=jax.ShapeDtypeStruct((NC, NP, HIDDEN), jnp.float32),
    scratch_types=[
        pltpu.VMEM((RPW1, K), jnp.int32),
        pltpu.VMEM((RPW1, K), jnp.int32),
        pltpu.VMEM((K, HIDDEN), jnp.float32),
        pltpu.SemaphoreType.DMA,
        pltpu.VMEM_SHARED((NP, HIDDEN), jnp.float32),
    ],
)
def _agg_sc(hs_hbm, src_hbm, dst_hbm, out_hbm, srcb, dstb, rows, sem, acc):
    c = lax.axis_index("c")
    s = lax.axis_index("s")
    wid = s * NC + c
    pltpu.sync_copy(src_hbm.at[pl.ds(wid * RPW1, RPW1)], srcb)
    pltpu.sync_copy(dst_hbm.at[pl.ds(wid * RPW1, RPW1)], dstb)
    base = s * RPT
    pltpu.sync_copy(hs_hbm.at[pl.ds(base, RPT)], acc.at[pl.ds(base, RPT)])
    plsc.subcore_barrier()

    def body(j, carry):
        pltpu.async_copy(hs_hbm.at[srcb.at[j]], rows, sem).wait()
        pltpu.sync_copy(rows, acc.at[dstb.at[j]], add=True)
        return carry

    lax.fori_loop(0, RPW1, body, 0)
    plsc.subcore_barrier()
    pltpu.sync_copy(acc.at[pl.ds(base, RPT)], out_hbm.at[c, pl.ds(base, RPT)])


@functools.partial(
    pl.kernel,
    mesh=_mesh,
    compiler_params=_sc_params,
    out_type=jax.ShapeDtypeStruct((EP2,), jnp.float32),
    scratch_types=[
        pltpu.VMEM((RPW2, K), jnp.int32),
        pltpu.VMEM((RPW2, K), jnp.int32),
        pltpu.VMEM((K, HIDDEN), jnp.float32),
        pltpu.VMEM((K, HIDDEN), jnp.float32),
        pltpu.VMEM((NP,), jnp.float32),
        pltpu.VMEM((EPW2,), jnp.float32),
        pltpu.SemaphoreType.DMA,
        pltpu.SemaphoreType.DMA,
    ],
)
def _dec_sc(q_hbm, a2_hbm, p_hbm, src_hbm, tar_hbm, out_hbm,
            srcb, tarb, qb, ab, pbuf, outbuf, sem1, sem2):
    c = lax.axis_index("c")
    s = lax.axis_index("s")
    wid = s * NC + c
    pltpu.sync_copy(src_hbm.at[pl.ds(wid * RPW2, RPW2)], srcb)
    pltpu.sync_copy(tar_hbm.at[pl.ds(wid * RPW2, RPW2)], tarb)
    pltpu.sync_copy(p_hbm, pbuf)
    iota = lax.iota(jnp.int32, 16)

    def body(j, carry):
        cp1 = pltpu.async_copy(q_hbm.at[srcb.at[j]], qb, sem1)
        cp2 = pltpu.async_copy(a2_hbm.at[tarb.at[j]], ab, sem2)
        cp1.wait()
        cp2.wait()
        for g in range(K // 16):
            rows = iota + (g * 16)
            sidx = srcb[j, pl.ds(g * 16, 16)]
            tidx = tarb[j, pl.ds(g * 16, 16)]
            acc = plsc.load_gather(pbuf, [sidx]) + plsc.load_gather(pbuf, [tidx])
            for d in range(HIDDEN):
                col = jnp.full((16,), d, jnp.int32)
                qv = plsc.load_gather(qb, [rows, col])
                av = plsc.load_gather(ab, [rows, col])
                acc = acc + qv * av
            outbuf[pl.ds(j * K + g * 16, 16)] = 1.0 / (1.0 + jnp.exp(-acc))
        return carry

    lax.fori_loop(0, RPW2, body, 0)
    pltpu.sync_copy(outbuf, out_hbm.at[pl.ds(wid * EPW2, EPW2)])


def _tc_a_body(x_ref, w1_ref, dacc_ref, hs_ref, dis_ref):
    deg = dacc_ref[0, :, 0:1] + dacc_ref[1, :, 0:1] + 1.0
    dis = lax.rsqrt(deg)
    h1 = jnp.dot(x_ref[...], w1_ref[...], preferred_element_type=jnp.float32)
    hs_ref[...] = h1 * dis
    dis_ref[...] = jnp.broadcast_to(dis, (NP, HIDDEN))


_tc_a = pl.pallas_call(
    _tc_a_body,
    out_shape=[jax.ShapeDtypeStruct((NP, HIDDEN), jnp.float32),
               jax.ShapeDtypeStruct((NP, HIDDEN), jnp.float32)],
)


def _tc_b_body(acc1_ref, hs_ref, dis_ref, b1_ref, rs_ref):
    dis = dis_ref[...]
    a1 = dis * (acc1_ref[0] + acc1_ref[1] - hs_ref[...])
    r = jnp.maximum(a1 + b1_ref[...], 0.0)
    rs_ref[...] = dis * r


_tc_b = pl.pallas_call(
    _tc_b_body,
    out_shape=[jax.ShapeDtypeStruct((NP, HIDDEN), jnp.float32)],
)


def _tc_c_body(acc2_ref, rs_ref, dis_ref, w2_ref, b2_ref, q_ref, a2_ref, p_ref):
    dis = dis_ref[...]
    a2 = dis * (acc2_ref[0] + acc2_ref[1] - rs_ref[...])
    w2 = w2_ref[...]
    b2 = b2_ref[...]
    g = lax.dot_general(w2, w2, (((1,), (1,)), ((), ())),
                        preferred_element_type=jnp.float32)
    u = lax.dot_general(w2, b2, (((1,), (1,)), ((), ())),
                        preferred_element_type=jnp.float32)
    c = jnp.sum(b2 * b2)
    q_ref[...] = jnp.dot(a2, g, preferred_element_type=jnp.float32)
    a2_ref[...] = a2
    p = jnp.dot(a2, u, preferred_element_type=jnp.float32) + 0.5 * c
    p_ref[...] = jnp.broadcast_to(p, (NP, HIDDEN))


_tc_c = pl.pallas_call(
    _tc_c_body,
    out_shape=[jax.ShapeDtypeStruct((NP, HIDDEN), jnp.float32),
               jax.ShapeDtypeStruct((NP, HIDDEN), jnp.float32),
               jax.ShapeDtypeStruct((NP, HIDDEN), jnp.float32)],
)


def kernel(x, pos_edge_index, neg_edge_index, W1, b1, W2, b2):
    i32 = jnp.int32
    ps = pos_edge_index.astype(i32)
    ns = neg_edge_index.astype(i32)

    pad1 = jnp.full((EP1 - E1,), SINK, i32)
    src1 = jnp.concatenate([ps[0], pad1]).reshape(STEPS1, K)
    dst1 = jnp.concatenate([ps[1], pad1]).reshape(STEPS1, K)
    x_ext = jnp.concatenate(
        [x, jnp.zeros((NP - N_NODES, D_FEAT), x.dtype)], axis=0)

    dacc = _deg_sc(dst1)
    hs, dis16 = _tc_a(x_ext, W1, dacc)
    acc1 = _agg_sc(hs, src1, dst1)
    (rs,) = _tc_b(acc1, hs, dis16, b1.reshape(1, HIDDEN))
    acc2 = _agg_sc(rs, src1, dst1)
    q, a2, p16 = _tc_c(acc2, rs, dis16, W2, b2.reshape(1, D_FEAT))

    pad2 = jnp.full((EP2 - E2,), SINK, i32)
    src2 = jnp.concatenate([ps[0], ns[0], pad2]).reshape(STEPS2, K)
    tar2 = jnp.concatenate([ps[1], ns[1], pad2]).reshape(STEPS2, K)
    logits = _dec_sc(q, a2, p16[:, 0], src2, tar2)
    return logits[:E2][:, None]
